# baseline (speedup 1.0000x reference)
import functools

import jax
import jax.numpy as jnp
import numpy as np
from jax import lax
from jax.experimental import pallas as pl
from jax.experimental.pallas import tpu as pltpu
from jax.experimental.pallas import tpu_sc as plsc

B, S, D, H, E, K = 4, 8192, 768, 768, 64, 2
T = B * S
CAP = int(round(K * T * 1.05 / E))
CP = 1152
EC = E * CP
TRASH = EC - 1
TBLK = 512
NB = T // TBLK

NC, NS = 2, 16
NW = NC * NS
CT = T // NW
SLOTS_PER_TILE = EC // NW

_SU = np.triu(np.ones((TBLK, TBLK), np.float32), k=1)


def _router_body(x_ref, wg_ref, e0_ref, e1_ref, v0_ref, v1_ref,
                 off0_ref, off1_ref, stats_ref, c0_s, c1_s, imp_s):
    b = pl.program_id(0)

    @pl.when(b == 0)
    def _():
        c0_s[...] = jnp.zeros((E, 128), jnp.float32)
        c1_s[...] = jnp.zeros((E, 128), jnp.float32)
        imp_s[...] = jnp.zeros((E, 128), jnp.float32)

    xb = x_ref[...]
    wg = wg_ref[...]
    logits = lax.dot_general(wg, xb, (((1,), (1,)), ((), ())),
                             preferred_element_type=jnp.float32)
    m = jnp.max(logits, axis=0, keepdims=True)
    ex = jnp.exp(logits - m)
    gates = ex / jnp.sum(ex, axis=0, keepdims=True)

    imp_s[...] += jnp.broadcast_to(jnp.sum(gates, axis=1, keepdims=True),
                                   (E, 128))

    iota_e = lax.broadcasted_iota(jnp.int32, (E, TBLK), 0)
    m1 = jnp.max(gates, axis=0, keepdims=True)
    e0 = jnp.min(jnp.where(gates == m1, iota_e, E), axis=0, keepdims=True)
    g2 = jnp.where(iota_e == e0, -1.0, gates)
    m2 = jnp.max(g2, axis=0, keepdims=True)
    e1 = jnp.min(jnp.where(g2 == m2, iota_e, E), axis=0, keepdims=True)

    oh0 = (iota_e == e0).astype(jnp.float32)
    oh1 = (iota_e == e1).astype(jnp.float32)
    cnt0 = jnp.sum(oh0, axis=1, keepdims=True)
    cnt1 = jnp.sum(oh1, axis=1, keepdims=True)

    e0_ref[...] = e0.reshape(1, 1, TBLK)
    e1_ref[...] = e1.reshape(1, 1, TBLK)
    v0_ref[...] = m1.reshape(1, 1, TBLK)
    v1_ref[...] = m2.reshape(1, 1, TBLK)
    off0_ref[...] = c0_s[...][None]
    off1_ref[...] = c1_s[...][None]
    c0_s[...] += jnp.broadcast_to(cnt0, (E, 128))
    c1_s[...] += jnp.broadcast_to(cnt1, (E, 128))
    stats_ref[...] = jnp.stack(
        [c0_s[...], c0_s[...] + c1_s[...], imp_s[...]], axis=0)


def _router(x2, Wg):
    return pl.pallas_call(
        _router_body,
        grid=(NB,),
        in_specs=[
            pl.BlockSpec((TBLK, D), lambda i: (i, 0)),
            pl.BlockSpec((E, D), lambda i: (0, 0)),
        ],
        out_specs=[
            pl.BlockSpec((1, 1, TBLK), lambda i: (i, 0, 0)),
            pl.BlockSpec((1, 1, TBLK), lambda i: (i, 0, 0)),
            pl.BlockSpec((1, 1, TBLK), lambda i: (i, 0, 0)),
            pl.BlockSpec((1, 1, TBLK), lambda i: (i, 0, 0)),
            pl.BlockSpec((1, E, 128), lambda i: (i, 0, 0)),
            pl.BlockSpec((1, E, 128), lambda i: (i, 0, 0)),
            pl.BlockSpec((3, E, 128), lambda i: (0, 0, 0)),
        ],
        out_shape=[
            jax.ShapeDtypeStruct((NB, 1, TBLK), jnp.int32),
            jax.ShapeDtypeStruct((NB, 1, TBLK), jnp.int32),
            jax.ShapeDtypeStruct((NB, 1, TBLK), jnp.float32),
            jax.ShapeDtypeStruct((NB, 1, TBLK), jnp.float32),
            jax.ShapeDtypeStruct((NB, E, 128), jnp.float32),
            jax.ShapeDtypeStruct((NB, E, 128), jnp.float32),
            jax.ShapeDtypeStruct((3, E, 128), jnp.float32),
        ],
        scratch_shapes=[pltpu.VMEM((E, 128), jnp.float32)] * 3,
    )(x2, Wg)


def _assign_body(e0_ref, e1_ref, v0_ref, v1_ref, off0_ref, off1_ref,
                 stats_ref, su_ref,
                 slot0_ref, slot1_ref, cw0_ref, cw1_ref, loss_ref):
    b = pl.program_id(0)
    e0 = e0_ref[0]
    e1 = e1_ref[0]
    v0 = v0_ref[0]
    v1 = v1_ref[0]
    su = su_ref[...]
    iota_e = lax.broadcasted_iota(jnp.int32, (E, TBLK), 0)

    oh0 = (iota_e == e0).astype(jnp.float32)
    oh1 = (iota_e == e1).astype(jnp.float32)
    rank0 = lax.dot_general(oh0, su, (((1,), (0,)), ((), ())),
                            preferred_element_type=jnp.float32)
    rank1 = lax.dot_general(oh1, su, (((1,), (0,)), ((), ())),
                            preferred_element_type=jnp.float32)
    off0 = off0_ref[0][:, :1]
    off1 = off1_ref[0][:, :1]
    total0 = stats_ref[0][:, :1]
    pos0 = jnp.sum(oh0 * (rank0 + off0), axis=0, keepdims=True)
    pos1 = jnp.sum(oh1 * (rank1 + off1 + total0), axis=0, keepdims=True)

    w0 = pos0 < CAP
    w1 = pos1 < CAP
    cw0 = v0 * w0.astype(jnp.float32)
    cw1 = v1 * w1.astype(jnp.float32)
    ssum = cw0 + cw1
    inv = 1.0 / jnp.maximum(ssum, 1e-30)
    cw0 = cw0 * inv
    cw1 = cw1 * inv

    slot0 = jnp.where(w0, e0 * CP + pos0.astype(jnp.int32), TRASH)
    slot1 = jnp.where(w1, e1 * CP + pos1.astype(jnp.int32), TRASH)
    slot0_ref[...] = slot0.reshape(1, 1, TBLK)
    slot1_ref[...] = slot1.reshape(1, 1, TBLK)
    cw0_ref[...] = cw0.reshape(1, 1, TBLK)
    cw1_ref[...] = cw1.reshape(1, 1, TBLK)

    @pl.when(b == NB - 1)
    def _():
        tot01 = stats_ref[1][:, :1]
        imp = stats_ref[2][:, :1]
        tpe = jnp.minimum(tot01, float(CAP))

        def cv2(v):
            mu = jnp.mean(v)
            var = jnp.mean((v - mu) ** 2)
            return var / (mu + 1e-6) ** 2

        l_imp = cv2(imp)
        l_load = cv2(tpe)
        l_aux = 0.5 * (l_imp + l_load)
        loss_ref[...] = jnp.concatenate(
            [jnp.full((1, 128), l_aux, jnp.float32),
             jnp.full((1, 128), l_load, jnp.float32),
             jnp.zeros((6, 128), jnp.float32)], axis=0)


def _assign(e0, e1, v0, v1, off0, off1, stats, su):
    return pl.pallas_call(
        _assign_body,
        grid=(NB,),
        in_specs=[
            pl.BlockSpec((1, 1, TBLK), lambda i: (i, 0, 0)),
            pl.BlockSpec((1, 1, TBLK), lambda i: (i, 0, 0)),
            pl.BlockSpec((1, 1, TBLK), lambda i: (i, 0, 0)),
            pl.BlockSpec((1, 1, TBLK), lambda i: (i, 0, 0)),
            pl.BlockSpec((1, E, 128), lambda i: (i, 0, 0)),
            pl.BlockSpec((1, E, 128), lambda i: (i, 0, 0)),
            pl.BlockSpec((3, E, 128), lambda i: (0, 0, 0)),
            pl.BlockSpec((TBLK, TBLK), lambda i: (0, 0)),
        ],
        out_specs=[
            pl.BlockSpec((1, 1, TBLK), lambda i: (i, 0, 0)),
            pl.BlockSpec((1, 1, TBLK), lambda i: (i, 0, 0)),
            pl.BlockSpec((1, 1, TBLK), lambda i: (i, 0, 0)),
            pl.BlockSpec((1, 1, TBLK), lambda i: (i, 0, 0)),
            pl.BlockSpec((8, 128), lambda i: (0, 0)),
        ],
        out_shape=[
            jax.ShapeDtypeStruct((NB, 1, TBLK), jnp.int32),
            jax.ShapeDtypeStruct((NB, 1, TBLK), jnp.int32),
            jax.ShapeDtypeStruct((NB, 1, TBLK), jnp.float32),
            jax.ShapeDtypeStruct((NB, 1, TBLK), jnp.float32),
            jax.ShapeDtypeStruct((8, 128), jnp.float32),
        ],
    )(e0, e1, v0, v1, off0, off1, stats, su)


_MESH = functools.partial(plsc.VectorSubcoreMesh, core_axis_name="c",
                          subcore_axis_name="s", num_cores=NC,
                          num_subcores=NS)


def _sc_wid():
    return lax.axis_index("s") * NC + lax.axis_index("c")


def _scatter_body(slot0_hbm, slot1_hbm, tab_hbm, tabp_v, idx0_v, idx1_v):
    wid = _sc_wid()
    zeros = jnp.zeros((16,), jnp.int32)

    def clear(i, _):
        tabp_v[pl.ds(i * 16, 16)] = zeros
        return 0

    lax.fori_loop(0, EC // 16, clear, 0)
    row0 = wid * 8
    pltpu.sync_copy(slot0_hbm.at[pl.ds(row0, 8)], idx0_v)
    pltpu.sync_copy(slot1_hbm.at[pl.ds(row0, 8)], idx1_v)
    base = wid * CT
    iota = lax.iota(jnp.int32, 16)

    def insert(s, t):
        wbase = (s // 16) * 16
        lane = s - wbase
        win = tabp_v[pl.ds(wbase, 16)]
        tabp_v[pl.ds(wbase, 16)] = jnp.where(iota == lane, t + 1, win)

    def group(c, _):
        j = c // 8
        l = c - j * 8
        s0 = idx0_v[j, pl.ds(l * 16, 16)]
        s1 = idx1_v[j, pl.ds(l * 16, 16)]
        t0 = base + c * 16
        for r in range(16):
            insert(s0[r], t0 + r)
        for r in range(16):
            insert(s1[r], t0 + r)
        return 0

    lax.fori_loop(0, CT // 16, group, 0)
    pltpu.sync_copy(tabp_v, tab_hbm.at[wid])


def _slot_scatter(slot0r, slot1r):
    return pl.kernel(
        _scatter_body,
        out_type=jax.ShapeDtypeStruct((NW, EC), jnp.int32),
        mesh=_MESH(),
        scratch_types=[
            pltpu.VMEM((EC,), jnp.int32),
            pltpu.VMEM((8, 128), jnp.int32),
            pltpu.VMEM((8, 128), jnp.int32),
        ],
    )(slot0r, slot1r)


def _gather_body(tab_hbm, x_hbm, xg_hbm, part_v, idx_v, rows_a, rows_b,
                 sem_ga, sem_gb, sem_w):
    wid = _sc_wid()
    s0 = wid * SLOTS_PER_TILE
    bufs = (rows_a, rows_b)
    sems = (sem_ga, sem_gb)

    def phase(p, _):
        r0 = pl.multiple_of(s0 + p * 128, 128)
        pltpu.sync_copy(tab_hbm.at[:, pl.ds(r0, 128)], part_v)
        for l in range(8):
            sl = pl.ds(l * 16, 16)
            m = part_v[0, sl]
            for w in range(1, NW):
                m = jnp.maximum(m, part_v[w, sl])
            idx_v[sl] = jnp.clip(m - 1, 0, T - 1)
        copies = []
        for q in range(2):
            @pl.when(p >= 1)
            def _(q=q):
                pltpu.make_async_copy(
                    xg_hbm.at[pl.ds(0, 64)], bufs[q], sem_w).wait()
            copies.append(
                pltpu.async_copy(x_hbm.at[idx_v.at[pl.ds(q * 64, 64)]],
                                 bufs[q], sems[q]))
        for q in range(2):
            copies[q].wait()
            pltpu.async_copy(bufs[q], xg_hbm.at[pl.ds(r0 + q * 64, 64)],
                             sem_w)
        return 0

    lax.fori_loop(0, SLOTS_PER_TILE // 128, phase, 0)
    for q in range(2):
        pltpu.make_async_copy(xg_hbm.at[pl.ds(0, 64)], bufs[q],
                              sem_w).wait()


def _row_gather(tab, xp):
    return pl.kernel(
        _gather_body,
        out_type=jax.ShapeDtypeStruct((EC, D // 2), jnp.float32),
        mesh=_MESH(),
        scratch_types=[
            pltpu.VMEM((NW, 128), jnp.int32),
            pltpu.VMEM((128,), jnp.int32),
            pltpu.VMEM((64, D // 2), jnp.float32),
            pltpu.VMEM((64, D // 2), jnp.float32),
            pltpu.SemaphoreType.DMA,
            pltpu.SemaphoreType.DMA,
            pltpu.SemaphoreType.DMA,
        ],
    )(tab, xp)


def _mlp_body(xg_ref, w1_ref, b1_ref, w2_ref, b2_ref, eo_ref):
    xe = xg_ref[0]
    w1 = w1_ref[0].astype(jnp.bfloat16)
    pre = lax.dot_general(xe, w1, (((1,), (1,)), ((), ())),
                          preferred_element_type=jnp.float32)
    pre = pre + b1_ref[0]
    h = 0.5 * pre * (1.0 + lax.erf(pre * (0.5 ** 0.5)))
    w2 = w2_ref[0].astype(jnp.bfloat16)
    out = lax.dot_general(h.astype(jnp.bfloat16), w2,
                          (((1,), (1,)), ((), ())),
                          preferred_element_type=jnp.float32)
    eo_ref[...] = (out + b2_ref[0])[None]


def _mlp(xg3, W1, b1, W2, b2):
    return pl.pallas_call(
        _mlp_body,
        grid=(E,),
        in_specs=[
            pl.BlockSpec((1, CP, D), lambda e: (e, 0, 0)),
            pl.BlockSpec((1, H, D), lambda e: (e, 0, 0)),
            pl.BlockSpec((1, 1, H), lambda e: (e, 0, 0)),
            pl.BlockSpec((1, D, H), lambda e: (e, 0, 0)),
            pl.BlockSpec((1, 1, D), lambda e: (e, 0, 0)),
        ],
        out_specs=pl.BlockSpec((1, CP, D), lambda e: (e, 0, 0)),
        out_shape=jax.ShapeDtypeStruct((E, CP, D), jnp.float32),
    )(xg3, W1, b1.reshape(E, 1, H), W2, b2.reshape(E, 1, D))


def _combine_body(eo_hbm, s0_hbm, s1_hbm, c0_hbm, c1_hbm, out_hbm,
                  s0_v, s1_v, c0_v, c1_v, g0_v, g1_v, o_v, cwb0_v, cwb1_v,
                  sem0, sem1):
    wid = _sc_wid()
    t0 = wid * CT
    pltpu.sync_copy(s0_hbm.at[pl.ds(t0, CT)], s0_v)
    pltpu.sync_copy(s1_hbm.at[pl.ds(t0, CT)], s1_v)
    pltpu.sync_copy(c0_hbm.at[pl.ds(t0, CT)], c0_v)
    pltpu.sync_copy(c1_hbm.at[pl.ds(t0, CT)], c1_v)

    def chunk(i, _):
        a0 = pltpu.async_copy(eo_hbm.at[s0_v.at[pl.ds(i * 32, 32)]], g0_v,
                              sem0)
        a1 = pltpu.async_copy(eo_hbm.at[s1_v.at[pl.ds(i * 32, 32)]], g1_v,
                              sem1)
        a0.wait()
        a1.wait()
        for g in range(2):
            cw0_vec = c0_v[pl.ds(i * 32 + g * 16, 16)]
            cw1_vec = c1_v[pl.ds(i * 32 + g * 16, 16)]
            for r in range(16):
                cwb0_v[r, :] = jnp.full((16,), cw0_vec[r], jnp.float32)
                cwb1_v[r, :] = jnp.full((16,), cw1_vec[r], jnp.float32)

            def row(r, _, g=g):
                rr = g * 16 + r
                b0 = cwb0_v[r, :]
                b1 = cwb1_v[r, :]
                for v in range(D // 16):
                    sl = pl.ds(v * 16, 16)
                    o_v[rr, sl] = g0_v[rr, sl] * b0 + g1_v[rr, sl] * b1
                return 0

            lax.fori_loop(0, 16, row, 0)
        pltpu.sync_copy(o_v, out_hbm.at[pl.ds(t0 + i * 32, 32)])
        return 0

    lax.fori_loop(0, CT // 32, chunk, 0)


def _combine(eo2, slot0, slot1, cw0, cw1):
    return pl.kernel(
        _combine_body,
        out_type=jax.ShapeDtypeStruct((T, D), jnp.float32),
        mesh=_MESH(),
        scratch_types=[
            pltpu.VMEM((CT,), jnp.int32),
            pltpu.VMEM((CT,), jnp.int32),
            pltpu.VMEM((CT,), jnp.float32),
            pltpu.VMEM((CT,), jnp.float32),
            pltpu.VMEM((32, D), jnp.float32),
            pltpu.VMEM((32, D), jnp.float32),
            pltpu.VMEM((32, D), jnp.float32),
            pltpu.VMEM((16, 16), jnp.float32),
            pltpu.VMEM((16, 16), jnp.float32),
            pltpu.SemaphoreType.DMA,
            pltpu.SemaphoreType.DMA,
        ],
    )(eo2, slot0, slot1, cw0, cw1)


def kernel(x, Wg, W1, b1, W2, b2):
    x2 = x.reshape(T, D)
    su = jnp.asarray(_SU)
    e0, e1, v0, v1, off0, off1, stats = _router(x2, Wg)
    slot0, slot1, cw0, cw1, losses = _assign(e0, e1, v0, v1, off0, off1,
                                             stats, su)
    tab = _slot_scatter(slot0.reshape(T // 128, 128),
                        slot1.reshape(T // 128, 128))
    xp = lax.bitcast_convert_type(
        x2.astype(jnp.bfloat16).reshape(T, D // 2, 2), jnp.float32)
    xg = _row_gather(tab, xp)
    xgb = lax.bitcast_convert_type(xg, jnp.bfloat16).reshape(EC, D)
    eo = _mlp(xgb.reshape(E, CP, D), W1, b1, W2, b2)
    out = _combine(eo.reshape(EC, D), slot0.reshape(T), slot1.reshape(T),
                   cw0.reshape(T), cw1.reshape(T))
    return (out.reshape(B, S, D), losses[0, 0], losses[1, 0])


# speedup vs baseline: 1.9863x; 1.9863x over previous
import functools

import jax
import jax.numpy as jnp
import numpy as np
from jax import lax
from jax.experimental import pallas as pl
from jax.experimental.pallas import tpu as pltpu
from jax.experimental.pallas import tpu_sc as plsc

B, S, D, H, E, K = 4, 8192, 768, 768, 64, 2
T = B * S
CAP = int(round(K * T * 1.05 / E))
CP = 1152
EC = E * CP
TRASH = EC - 1
TBLK = 512
NB = T // TBLK

NC, NS = 2, 16
NW = NC * NS
CT = T // NW
SLOTS_PER_TILE = EC // NW

_SU = np.triu(np.ones((TBLK, TBLK), np.float32), k=1)


def _router_body(x_ref, wg_ref, e0_ref, e1_ref, v0_ref, v1_ref,
                 off0_ref, off1_ref, stats_ref, c0_s, c1_s, imp_s):
    b = pl.program_id(0)

    @pl.when(b == 0)
    def _():
        c0_s[...] = jnp.zeros((E, 128), jnp.float32)
        c1_s[...] = jnp.zeros((E, 128), jnp.float32)
        imp_s[...] = jnp.zeros((E, 128), jnp.float32)

    xb = x_ref[...]
    wg = wg_ref[...]
    logits = lax.dot_general(wg, xb, (((1,), (1,)), ((), ())),
                             preferred_element_type=jnp.float32)
    m = jnp.max(logits, axis=0, keepdims=True)
    ex = jnp.exp(logits - m)
    gates = ex / jnp.sum(ex, axis=0, keepdims=True)

    imp_s[...] += jnp.broadcast_to(jnp.sum(gates, axis=1, keepdims=True),
                                   (E, 128))

    iota_e = lax.broadcasted_iota(jnp.int32, (E, TBLK), 0)
    m1 = jnp.max(gates, axis=0, keepdims=True)
    e0 = jnp.min(jnp.where(gates == m1, iota_e, E), axis=0, keepdims=True)
    g2 = jnp.where(iota_e == e0, -1.0, gates)
    m2 = jnp.max(g2, axis=0, keepdims=True)
    e1 = jnp.min(jnp.where(g2 == m2, iota_e, E), axis=0, keepdims=True)

    oh0 = (iota_e == e0).astype(jnp.float32)
    oh1 = (iota_e == e1).astype(jnp.float32)
    cnt0 = jnp.sum(oh0, axis=1, keepdims=True)
    cnt1 = jnp.sum(oh1, axis=1, keepdims=True)

    e0_ref[...] = e0.reshape(1, 1, TBLK)
    e1_ref[...] = e1.reshape(1, 1, TBLK)
    v0_ref[...] = m1.reshape(1, 1, TBLK)
    v1_ref[...] = m2.reshape(1, 1, TBLK)
    off0_ref[...] = c0_s[...][None]
    off1_ref[...] = c1_s[...][None]
    c0_s[...] += jnp.broadcast_to(cnt0, (E, 128))
    c1_s[...] += jnp.broadcast_to(cnt1, (E, 128))
    stats_ref[...] = jnp.stack(
        [c0_s[...], c0_s[...] + c1_s[...], imp_s[...]], axis=0)


def _router(x2, Wg):
    return pl.pallas_call(
        _router_body,
        grid=(NB,),
        in_specs=[
            pl.BlockSpec((TBLK, D), lambda i: (i, 0)),
            pl.BlockSpec((E, D), lambda i: (0, 0)),
        ],
        out_specs=[
            pl.BlockSpec((1, 1, TBLK), lambda i: (i, 0, 0)),
            pl.BlockSpec((1, 1, TBLK), lambda i: (i, 0, 0)),
            pl.BlockSpec((1, 1, TBLK), lambda i: (i, 0, 0)),
            pl.BlockSpec((1, 1, TBLK), lambda i: (i, 0, 0)),
            pl.BlockSpec((1, E, 128), lambda i: (i, 0, 0)),
            pl.BlockSpec((1, E, 128), lambda i: (i, 0, 0)),
            pl.BlockSpec((3, E, 128), lambda i: (0, 0, 0)),
        ],
        out_shape=[
            jax.ShapeDtypeStruct((NB, 1, TBLK), jnp.int32),
            jax.ShapeDtypeStruct((NB, 1, TBLK), jnp.int32),
            jax.ShapeDtypeStruct((NB, 1, TBLK), jnp.float32),
            jax.ShapeDtypeStruct((NB, 1, TBLK), jnp.float32),
            jax.ShapeDtypeStruct((NB, E, 128), jnp.float32),
            jax.ShapeDtypeStruct((NB, E, 128), jnp.float32),
            jax.ShapeDtypeStruct((3, E, 128), jnp.float32),
        ],
        scratch_shapes=[pltpu.VMEM((E, 128), jnp.float32)] * 3,
    )(x2, Wg)


def _assign_body(e0_ref, e1_ref, v0_ref, v1_ref, off0_ref, off1_ref,
                 stats_ref, su_ref,
                 slot0_ref, slot1_ref, cw0_ref, cw1_ref, loss_ref):
    b = pl.program_id(0)
    e0 = e0_ref[0]
    e1 = e1_ref[0]
    v0 = v0_ref[0]
    v1 = v1_ref[0]
    su = su_ref[...]
    iota_e = lax.broadcasted_iota(jnp.int32, (E, TBLK), 0)

    oh0 = (iota_e == e0).astype(jnp.float32)
    oh1 = (iota_e == e1).astype(jnp.float32)
    rank0 = lax.dot_general(oh0, su, (((1,), (0,)), ((), ())),
                            preferred_element_type=jnp.float32)
    rank1 = lax.dot_general(oh1, su, (((1,), (0,)), ((), ())),
                            preferred_element_type=jnp.float32)
    off0 = off0_ref[0][:, :1]
    off1 = off1_ref[0][:, :1]
    total0 = stats_ref[0][:, :1]
    pos0 = jnp.sum(oh0 * (rank0 + off0), axis=0, keepdims=True)
    pos1 = jnp.sum(oh1 * (rank1 + off1 + total0), axis=0, keepdims=True)

    w0 = pos0 < CAP
    w1 = pos1 < CAP
    cw0 = v0 * w0.astype(jnp.float32)
    cw1 = v1 * w1.astype(jnp.float32)
    ssum = cw0 + cw1
    inv = 1.0 / jnp.maximum(ssum, 1e-30)
    cw0 = cw0 * inv
    cw1 = cw1 * inv

    slot0 = jnp.where(w0, e0 * CP + pos0.astype(jnp.int32), TRASH)
    slot1 = jnp.where(w1, e1 * CP + pos1.astype(jnp.int32), TRASH)
    slot0_ref[...] = slot0.reshape(1, 1, TBLK)
    slot1_ref[...] = slot1.reshape(1, 1, TBLK)
    cw0_ref[...] = cw0.reshape(1, 1, TBLK)
    cw1_ref[...] = cw1.reshape(1, 1, TBLK)

    @pl.when(b == NB - 1)
    def _():
        tot01 = stats_ref[1][:, :1]
        imp = stats_ref[2][:, :1]
        tpe = jnp.minimum(tot01, float(CAP))

        def cv2(v):
            mu = jnp.mean(v)
            var = jnp.mean((v - mu) ** 2)
            return var / (mu + 1e-6) ** 2

        l_imp = cv2(imp)
        l_load = cv2(tpe)
        l_aux = 0.5 * (l_imp + l_load)
        loss_ref[...] = jnp.concatenate(
            [jnp.full((1, 128), l_aux, jnp.float32),
             jnp.full((1, 128), l_load, jnp.float32),
             jnp.zeros((6, 128), jnp.float32)], axis=0)


def _assign(e0, e1, v0, v1, off0, off1, stats, su):
    return pl.pallas_call(
        _assign_body,
        grid=(NB,),
        in_specs=[
            pl.BlockSpec((1, 1, TBLK), lambda i: (i, 0, 0)),
            pl.BlockSpec((1, 1, TBLK), lambda i: (i, 0, 0)),
            pl.BlockSpec((1, 1, TBLK), lambda i: (i, 0, 0)),
            pl.BlockSpec((1, 1, TBLK), lambda i: (i, 0, 0)),
            pl.BlockSpec((1, E, 128), lambda i: (i, 0, 0)),
            pl.BlockSpec((1, E, 128), lambda i: (i, 0, 0)),
            pl.BlockSpec((3, E, 128), lambda i: (0, 0, 0)),
            pl.BlockSpec((TBLK, TBLK), lambda i: (0, 0)),
        ],
        out_specs=[
            pl.BlockSpec((1, 1, TBLK), lambda i: (i, 0, 0)),
            pl.BlockSpec((1, 1, TBLK), lambda i: (i, 0, 0)),
            pl.BlockSpec((1, 1, TBLK), lambda i: (i, 0, 0)),
            pl.BlockSpec((1, 1, TBLK), lambda i: (i, 0, 0)),
            pl.BlockSpec((8, 128), lambda i: (0, 0)),
        ],
        out_shape=[
            jax.ShapeDtypeStruct((NB, 1, TBLK), jnp.int32),
            jax.ShapeDtypeStruct((NB, 1, TBLK), jnp.int32),
            jax.ShapeDtypeStruct((NB, 1, TBLK), jnp.float32),
            jax.ShapeDtypeStruct((NB, 1, TBLK), jnp.float32),
            jax.ShapeDtypeStruct((8, 128), jnp.float32),
        ],
    )(e0, e1, v0, v1, off0, off1, stats, su)


_MESH = functools.partial(plsc.VectorSubcoreMesh, core_axis_name="c",
                          subcore_axis_name="s", num_cores=NC,
                          num_subcores=NS)


def _sc_wid():
    return lax.axis_index("s") * NC + lax.axis_index("c")


def _scatter_body(slot0_hbm, slot1_hbm, tab_hbm, tabp_v, idx0_v, idx1_v):
    wid = _sc_wid()
    zeros = jnp.zeros((16,), jnp.int32)

    def clear(i, _):
        tabp_v[pl.ds(i * 16, 16)] = zeros
        return 0

    lax.fori_loop(0, EC // 16, clear, 0)
    row0 = wid * 8
    pltpu.sync_copy(slot0_hbm.at[pl.ds(row0, 8)], idx0_v)
    pltpu.sync_copy(slot1_hbm.at[pl.ds(row0, 8)], idx1_v)
    base = wid * CT
    iota = lax.iota(jnp.int32, 16)

    def insert(s, t):
        wbase = (s // 16) * 16
        lane = s - wbase
        win = tabp_v[pl.ds(wbase, 16)]
        tabp_v[pl.ds(wbase, 16)] = jnp.where(iota == lane, t + 1, win)

    def group(c, _):
        j = c // 8
        l = c - j * 8
        s0 = idx0_v[j, pl.ds(l * 16, 16)]
        s1 = idx1_v[j, pl.ds(l * 16, 16)]
        t0 = base + c * 16
        for r in range(16):
            insert(s0[r], t0 + r)
        for r in range(16):
            insert(s1[r], t0 + r)
        return 0

    lax.fori_loop(0, CT // 16, group, 0)
    pltpu.sync_copy(tabp_v, tab_hbm.at[wid])


def _slot_scatter(slot0r, slot1r):
    return pl.kernel(
        _scatter_body,
        out_type=jax.ShapeDtypeStruct((NW, EC), jnp.int32),
        mesh=_MESH(),
        scratch_types=[
            pltpu.VMEM((EC,), jnp.int32),
            pltpu.VMEM((8, 128), jnp.int32),
            pltpu.VMEM((8, 128), jnp.int32),
        ],
    )(slot0r, slot1r)


def _gather_body(tab_hbm, x_hbm, xg_hbm, part_v, idx_v, rows_a, rows_b,
                 sem_ga, sem_gb, sem_w):
    wid = _sc_wid()
    s0 = wid * SLOTS_PER_TILE
    bufs = (rows_a, rows_b)
    sems = (sem_ga, sem_gb)

    def phase(p, _):
        r0 = pl.multiple_of(s0 + p * 128, 128)
        pltpu.sync_copy(tab_hbm.at[:, pl.ds(r0, 128)], part_v)
        for l in range(8):
            sl = pl.ds(l * 16, 16)
            m = part_v[0, sl]
            for w in range(1, NW):
                m = jnp.maximum(m, part_v[w, sl])
            idx_v[sl] = jnp.clip(m - 1, 0, T - 1)
        copies = []
        for q in range(2):
            @pl.when(p >= 1)
            def _(q=q):
                pltpu.make_async_copy(
                    xg_hbm.at[pl.ds(0, 64)], bufs[q], sem_w).wait()
            copies.append(
                pltpu.async_copy(x_hbm.at[idx_v.at[pl.ds(q * 64, 64)]],
                                 bufs[q], sems[q]))
        for q in range(2):
            copies[q].wait()
            pltpu.async_copy(bufs[q], xg_hbm.at[pl.ds(r0 + q * 64, 64)],
                             sem_w)
        return 0

    lax.fori_loop(0, SLOTS_PER_TILE // 128, phase, 0)
    for q in range(2):
        pltpu.make_async_copy(xg_hbm.at[pl.ds(0, 64)], bufs[q],
                              sem_w).wait()


def _row_gather(tab, x2):
    return pl.kernel(
        _gather_body,
        out_type=jax.ShapeDtypeStruct((EC, D), jnp.float32),
        mesh=_MESH(),
        scratch_types=[
            pltpu.VMEM((NW, 128), jnp.int32),
            pltpu.VMEM((128,), jnp.int32),
            pltpu.VMEM((64, D), jnp.float32),
            pltpu.VMEM((64, D), jnp.float32),
            pltpu.SemaphoreType.DMA,
            pltpu.SemaphoreType.DMA,
            pltpu.SemaphoreType.DMA,
        ],
    )(tab, x2)


def _mlp_body(xg_ref, w1_ref, b1_ref, w2_ref, b2_ref, eo_ref):
    xe = xg_ref[0].astype(jnp.bfloat16)
    w1 = w1_ref[0].astype(jnp.bfloat16)
    pre = lax.dot_general(xe, w1, (((1,), (1,)), ((), ())),
                          preferred_element_type=jnp.float32)
    pre = pre + b1_ref[0]
    h = 0.5 * pre * (1.0 + lax.erf(pre * (0.5 ** 0.5)))
    w2 = w2_ref[0].astype(jnp.bfloat16)
    out = lax.dot_general(h.astype(jnp.bfloat16), w2,
                          (((1,), (1,)), ((), ())),
                          preferred_element_type=jnp.float32)
    eo_ref[...] = (out + b2_ref[0])[None]


def _mlp(xg3, W1, b1, W2, b2):
    return pl.pallas_call(
        _mlp_body,
        grid=(E,),
        in_specs=[
            pl.BlockSpec((1, CP, D), lambda e: (e, 0, 0)),
            pl.BlockSpec((1, H, D), lambda e: (e, 0, 0)),
            pl.BlockSpec((1, 1, H), lambda e: (e, 0, 0)),
            pl.BlockSpec((1, D, H), lambda e: (e, 0, 0)),
            pl.BlockSpec((1, 1, D), lambda e: (e, 0, 0)),
        ],
        out_specs=pl.BlockSpec((1, CP, D), lambda e: (e, 0, 0)),
        out_shape=jax.ShapeDtypeStruct((E, CP, D), jnp.float32),
    )(xg3, W1, b1.reshape(E, 1, H), W2, b2.reshape(E, 1, D))


def _combine_body(eo_hbm, s0_hbm, s1_hbm, c0_hbm, c1_hbm, out_hbm,
                  s0_v, s1_v, c0_v, c1_v, g0_v, g1_v, o_v, cwb0_v, cwb1_v,
                  sem0, sem1, sem_w):
    wid = _sc_wid()
    t0 = wid * CT
    pltpu.sync_copy(s0_hbm.at[pl.ds(t0, CT)], s0_v)
    pltpu.sync_copy(s1_hbm.at[pl.ds(t0, CT)], s1_v)
    pltpu.sync_copy(c0_hbm.at[pl.ds(t0, CT)], c0_v)
    pltpu.sync_copy(c1_hbm.at[pl.ds(t0, CT)], c1_v)

    NCH = CT // 16

    def fire(c, s):
        pltpu.async_copy(eo_hbm.at[s0_v.at[pl.ds(c * 16, 16)]], g0_v[s],
                         sem0[s])
        pltpu.async_copy(eo_hbm.at[s1_v.at[pl.ds(c * 16, 16)]], g1_v[s],
                         sem1[s])

    fire(0, 0)

    def phase(p, _):
        for s in range(2):
            c = p * 2 + s
            pltpu.make_async_copy(eo_hbm.at[pl.ds(0, 16)], g0_v[s],
                                  sem0[s]).wait()
            pltpu.make_async_copy(eo_hbm.at[pl.ds(0, 16)], g1_v[s],
                                  sem1[s]).wait()

            @pl.when(c + 1 < NCH)
            def _(c=c, s=s):
                fire(c + 1, 1 - s)

            @pl.when(c >= 2)
            def _(s=s):
                pltpu.make_async_copy(out_hbm.at[pl.ds(0, 16)], o_v[s],
                                      sem_w).wait()
            cw0_vec = c0_v[pl.ds(c * 16, 16)]
            cw1_vec = c1_v[pl.ds(c * 16, 16)]
            for r in range(16):
                cwb0_v[r, :] = jnp.full((16,), cw0_vec[r], jnp.float32)
                cwb1_v[r, :] = jnp.full((16,), cw1_vec[r], jnp.float32)

            def row(r, _, s=s):
                b0 = cwb0_v[r, :]
                b1 = cwb1_v[r, :]
                for v in range(D // 16):
                    sl = pl.ds(v * 16, 16)
                    o_v[s][r, sl] = (g0_v[s][r, sl] * b0 +
                                     g1_v[s][r, sl] * b1)
                return 0

            lax.fori_loop(0, 16, row, 0)
            pltpu.async_copy(o_v[s], out_hbm.at[pl.ds(t0 + c * 16, 16)],
                             sem_w)
        return 0

    lax.fori_loop(0, NCH // 2, phase, 0)
    for s in range(2):
        pltpu.make_async_copy(out_hbm.at[pl.ds(0, 16)], o_v[s], sem_w).wait()


def _combine_entry(eo_hbm, s0_hbm, s1_hbm, c0_hbm, c1_hbm, out_hbm,
                   s0_v, s1_v, c0_v, c1_v, g0a, g0b, g1a, g1b, oa, ob,
                   cwb0_v, cwb1_v, sem0a, sem0b, sem1a, sem1b, sem_w):
    _combine_body(eo_hbm, s0_hbm, s1_hbm, c0_hbm, c1_hbm, out_hbm,
                  s0_v, s1_v, c0_v, c1_v, (g0a, g0b), (g1a, g1b), (oa, ob),
                  cwb0_v, cwb1_v, (sem0a, sem0b), (sem1a, sem1b), sem_w)


def _combine(eo2, slot0, slot1, cw0, cw1):
    return pl.kernel(
        _combine_entry,
        out_type=jax.ShapeDtypeStruct((T, D), jnp.float32),
        mesh=_MESH(),
        scratch_types=[
            pltpu.VMEM((CT,), jnp.int32),
            pltpu.VMEM((CT,), jnp.int32),
            pltpu.VMEM((CT,), jnp.float32),
            pltpu.VMEM((CT,), jnp.float32),
            pltpu.VMEM((16, D), jnp.float32),
            pltpu.VMEM((16, D), jnp.float32),
            pltpu.VMEM((16, D), jnp.float32),
            pltpu.VMEM((16, D), jnp.float32),
            pltpu.VMEM((16, D), jnp.float32),
            pltpu.VMEM((16, D), jnp.float32),
            pltpu.VMEM((16, 16), jnp.float32),
            pltpu.VMEM((16, 16), jnp.float32),
            pltpu.SemaphoreType.DMA,
            pltpu.SemaphoreType.DMA,
            pltpu.SemaphoreType.DMA,
            pltpu.SemaphoreType.DMA,
            pltpu.SemaphoreType.DMA,
        ],
    )(eo2, slot0, slot1, cw0, cw1)


def kernel(x, Wg, W1, b1, W2, b2):
    x2 = x.reshape(T, D)
    su = jnp.asarray(_SU)
    e0, e1, v0, v1, off0, off1, stats = _router(x2, Wg)
    slot0, slot1, cw0, cw1, losses = _assign(e0, e1, v0, v1, off0, off1,
                                             stats, su)
    tab = _slot_scatter(slot0.reshape(T // 128, 128),
                        slot1.reshape(T // 128, 128))
    xg = _row_gather(tab, x2)
    eo = _mlp(xg.reshape(E, CP, D), W1, b1, W2, b2)
    out = _combine(eo.reshape(EC, D), slot0.reshape(T), slot1.reshape(T),
                   cw0.reshape(T), cw1.reshape(T))
    return (out.reshape(B, S, D), losses[0, 0], losses[1, 0])
